# Initial kernel scaffold; baseline (speedup 1.0000x reference)
import jax
import jax.numpy as jnp
from jax.experimental import pallas as pl


def kernel(full_dofs, kin_id, pose_coords):
    raise NotImplementedError("write your pallas kernel here")


# speedup vs baseline: 38.4421x; 38.4421x over previous
import functools

import jax
import jax.numpy as jnp
from jax import lax
from jax.experimental import pallas as pl
from jax.experimental.pallas import tpu as pltpu
from jax.experimental.pallas import tpu_sc as plsc

_N_POSES = 64
_APP = 4096
_M = _N_POSES * _APP
_N = 262144

_C = 128
_R = _N // _C

_SENT = jnp.int32(0x3FFFFFFF)

_IDENT = (1.0, 0.0, 0.0, 0.0, 1.0, 0.0, 0.0, 0.0, 1.0, 0.0, 0.0, 0.0)


def _round_bf16(comps):
    return tuple(c.astype(jnp.bfloat16).astype(jnp.float32) for c in comps)


def _compose(a, b):
    a00, a01, a02, a10, a11, a12, a20, a21, a22, ax, ay, az = a
    b00, b01, b02, b10, b11, b12, b20, b21, b22, bx, by, bz = b

    def d3(p, q, r, s, t, u):
        return (p * q + r * s) + t * u

    def d3t(p, q, r, s, t, u, v):
        return (p * q + r * s) + (t * u + v)

    return (
        d3(a00, b00, a01, b10, a02, b20),
        d3(a00, b01, a01, b11, a02, b21),
        d3(a00, b02, a01, b12, a02, b22),
        d3(a10, b00, a11, b10, a12, b20),
        d3(a10, b01, a11, b11, a12, b21),
        d3(a10, b02, a11, b12, a12, b22),
        d3(a20, b00, a21, b10, a22, b20),
        d3(a20, b01, a21, b11, a22, b21),
        d3(a20, b02, a21, b12, a22, b22),
        d3t(a00, bx, a01, by, a02, bz, ax),
        d3t(a10, bx, a11, by, a12, bz, ay),
        d3t(a20, bx, a21, by, a22, bz, az),
    )


def _sel_mat(rows, cols, stride, offset):
    i0 = lax.broadcasted_iota(jnp.int32, (rows, cols), 0)
    i1 = lax.broadcasted_iota(jnp.int32, (rows, cols), 1)
    return (i0 == stride * i1 + offset).astype(jnp.float32)


def _dot_hi(a, b):
    return lax.dot_general(a, b, (((1,), (0,)), ((), ())),
                           precision=lax.Precision.HIGHEST)


def _deinterleave(comps, m):
    if m >= 256:
        x2 = [c.reshape(c.shape[0] // 2, 2, 128) for c in comps]
        return (tuple(c[:, 0, :] for c in x2), tuple(c[:, 1, :] for c in x2))
    if m == 2:
        return (tuple(c[:, 0:1] for c in comps),
                tuple(c[:, 1:2] for c in comps))
    qe = _sel_mat(m, m // 2, 2, 0)
    qo = _sel_mat(m, m // 2, 2, 1)
    return (tuple(_dot_hi(c, qe) for c in comps),
            tuple(_dot_hi(c, qo) for c in comps))


def _interleave(ev, od, m):
    if m >= 256:
        return tuple(
            jnp.stack([e, o], axis=1).reshape(e.shape[0] * 2, 128)
            for e, o in zip(ev, od))
    if m == 2:
        return tuple(jnp.concatenate([e, o], axis=1) for e, o in zip(ev, od))
    pe = _sel_mat(m, m // 2, 2, 0).T
    po = _sel_mat(m, m // 2, 2, 1).T
    return tuple(_dot_hi(e, pe) + _dot_hi(o, po) for e, o in zip(ev, od))


def _shift1(comps, m):
    out = []
    for i, x in enumerate(comps):
        if m >= 256:
            top = jnp.concatenate(
                [jnp.full((1, 1), _IDENT[i], x.dtype),
                 x[x.shape[0] - 1:, : _C - 1]], axis=1)
            out.append(jnp.concatenate([top, x[:-1, :]], axis=0))
        else:
            out.append(jnp.concatenate(
                [jnp.full((1, 1), _IDENT[i], x.dtype), x[:, : m - 1]],
                axis=1))
    return tuple(out)


def _set_elem0(comps, f0):
    i0 = lax.broadcasted_iota(jnp.int32, comps[0].shape, 0)
    i1 = lax.broadcasted_iota(jnp.int32, comps[0].shape, 1)
    mask = (i0 == 0) & (i1 == 0)
    return tuple(jnp.where(mask, f, c) for c, f in zip(comps, f0))


_SUBLANE_LVLS = 11
_CHR = 64


def _scan_body(dofs_ref, pc_ref, snorm_ref, pnorm_ref, *scr):
    be_refs = scr[:_SUBLANE_LVLS]
    b_a, b_b, s_a, s_b, f_ref = scr[_SUBLANE_LVLS:]
    b_pp = (b_a, b_b)
    s_pp = (s_a, s_b)

    def _h_comps(d, th, ph):
        ct = jnp.cos(th)
        st = jnp.sin(th)
        cp = jnp.cos(ph)
        sp = jnp.sin(ph)
        return (cp * ct, -sp, cp * st,
                sp * ct, cp, sp * st,
                -st, jnp.zeros_like(d), ct,
                (cp * ct) * d, (sp * ct) * d, (-st) * d)

    def _build(k, _):
        sl = pl.ds(k * _CHR, _CHR)
        h = _h_comps(dofs_ref[0, sl, :] * 0.01 + 1.5,
                     dofs_ref[1, sl, :], dofs_ref[2, sl, :])
        for i, c in enumerate(h):
            b_a[i, sl, :] = c.astype(jnp.bfloat16)
        p0 = pc_ref[0, sl, :]
        p1 = pc_ref[1, sl, :]
        p2 = pc_ref[2, sl, :]
        pnorm_ref[sl, :] = p0 * p0 + p1 * p1 + p2 * p2
        return 0

    lax.fori_loop(0, _R // _CHR, _build, 0)
    h0 = _h_comps(dofs_ref[0, 0:1, 0:1] * 0.01 + 1.5,
                  dofs_ref[1, 0:1, 0:1], dofs_ref[2, 0:1, 0:1])
    for i in range(12):
        f_ref[0, i] = h0[i]

    for lvl in range(_SUBLANE_LVLS):
        xh = max(1, _R >> (lvl + 1))
        src = b_pp[lvl % 2]
        dst = b_pp[(lvl + 1) % 2]

        def _down(k, _, src=src, dst=dst, lvl=lvl):
            ch = min(_CHR, xh)
            r0 = k * ch
            be, bo = [], []
            for i in range(12):
                x = src[i, pl.ds(2 * r0, 2 * ch), :].astype(jnp.float32)
                x2 = x.reshape(ch, 2, _C)
                be.append(x2[:, 0, :])
                bo.append(x2[:, 1, :])
                be_refs[lvl][i, pl.ds(r0, ch), :] = be[i].astype(jnp.bfloat16)
            c = _compose(tuple(be), tuple(bo))
            for i in range(12):
                dst[i, pl.ds(r0, ch), :] = c[i].astype(jnp.bfloat16)
            return 0

        if xh <= _CHR:
            _down(0, 0)
        else:
            lax.fori_loop(0, xh // _CHR, _down, 0)
        b0 = tuple(src[i, 0:1, 0:1].astype(jnp.float32) for i in range(12))
        b1 = tuple(src[i, 1:2, 0:1].astype(jnp.float32) for i in range(12))
        c0 = _compose(b0, b1)
        for i in range(12):
            f_ref[lvl + 1, i] = c0[i]

    b = tuple(b_pp[_SUBLANE_LVLS % 2][i, 0:1, :].astype(jnp.float32)
              for i in range(12))
    lane_be = {}
    for lvl in range(_SUBLANE_LVLS, 17):
        m = _N >> lvl
        be, bo = _deinterleave(b, m)
        lane_be[lvl] = be
        c = _compose(be, bo)
        for i in range(12):
            f_ref[lvl + 1, i] = c[i][0:1, 0:1]
        b = _round_bf16(c)
    be, bo = _deinterleave(b, 2)
    lane_be[17] = be
    c18 = _compose(be, bo)

    s = c18
    for lvl in range(17, _SUBLANE_LVLS - 1, -1):
        m = _N >> lvl
        if m == 2:
            ev = tuple(f_ref[17, i] for i in range(12))
        else:
            z = _round_bf16(_shift1(s, m // 2))
            ev = _compose(z, lane_be[lvl])
            ev = _set_elem0(ev, tuple(f_ref[lvl, i] for i in range(12)))
        s = _interleave(ev, s, m)
    for i in range(12):
        s_pp[_SUBLANE_LVLS % 2][i, 0:1, :] = s[i]

    for lvl in range(_SUBLANE_LVLS - 1, -1, -1):
        xh = max(1, _R >> (lvl + 1))
        src = s_pp[(lvl + 1) % 2]
        ch = min(_CHR, xh)
        carry0 = tuple(
            jnp.concatenate(
                [jnp.full((1, 1), _IDENT[i], jnp.float32),
                 src[i, xh - 1:xh, :_C - 1]], axis=1)
            for i in range(12))

        def _zchunk(a, carry):
            if ch == 1:
                return carry
            return tuple(
                jnp.concatenate([carry[i], a[i][:ch - 1, :]], axis=0)
                for i in range(12))

        if lvl > 0:
            dst = s_pp[lvl % 2]

            def _up(k, carry, src=src, dst=dst, lvl=lvl, ch=ch):
                r0 = k * ch
                a = tuple(src[i, pl.ds(r0, ch), :] for i in range(12))
                z = _round_bf16(_zchunk(a, carry))
                bev = tuple(
                    be_refs[lvl][i, pl.ds(r0, ch), :].astype(jnp.float32)
                    for i in range(12))
                ev = _compose(z, bev)
                for i in range(12):
                    dst[i, pl.ds(2 * r0, 2 * ch), :] = (
                        jnp.stack([ev[i], a[i]], axis=1).reshape(2 * ch, _C))
                return tuple(a[i][ch - 1:ch, :] for i in range(12))

            if xh <= _CHR:
                _up(0, carry0)
            else:
                lax.fori_loop(0, xh // _CHR, _up, carry0)
            for i in range(12):
                dst[i, 0:1, 0:1] = f_ref[lvl, i]
        else:
            def _fin(k, carry):
                r0 = k * _CHR
                a = tuple(src[i, pl.ds(r0, _CHR), :] for i in range(12))
                z = _round_bf16(_zchunk(a, carry))
                z00, z01, z02, z10, z11, z12, z20, z21, z22, zx, zy, zz = z
                bx = be_refs[0][9, pl.ds(r0, _CHR), :].astype(jnp.float32)
                by = be_refs[0][10, pl.ds(r0, _CHR), :].astype(jnp.float32)
                bz = be_refs[0][11, pl.ds(r0, _CHR), :].astype(jnp.float32)
                ex = (z00 * bx + z01 * by) + (z02 * bz + zx)
                ey = (z10 * bx + z11 * by) + (z12 * bz + zy)
                ez = (z20 * bx + z21 * by) + (z22 * bz + zz)
                sn_e = ex * ex + ey * ey + ez * ez
                sn_o = a[9] * a[9] + a[10] * a[10] + a[11] * a[11]
                snorm_ref[pl.ds(2 * r0, 2 * _CHR), :] = (
                    jnp.stack([sn_e, sn_o], axis=1).reshape(2 * _CHR, _C))
                return tuple(a[i][_CHR - 1:_CHR, :] for i in range(12))

            lax.fori_loop(0, xh // _CHR, _fin, carry0)
            fx = f_ref[0, 9]
            fy = f_ref[0, 10]
            fz = f_ref[0, 11]
            snorm_ref[0:1, 0:1] = fx * fx + fy * fy + fz * fz


def _scan_tc(dofs_cm, pc_t):
    scratch = [
        pltpu.VMEM((12, max(1, _R >> (lvl + 1)), _C), jnp.bfloat16)
        for lvl in range(_SUBLANE_LVLS)
    ]
    scratch.append(pltpu.VMEM((12, _R, _C), jnp.bfloat16))
    scratch.append(pltpu.VMEM((12, _R // 2, _C), jnp.bfloat16))
    scratch.append(pltpu.VMEM((12, _R // 4, _C), jnp.float32))
    scratch.append(pltpu.VMEM((12, _R // 2, _C), jnp.float32))
    scratch.append(pltpu.VMEM((19, 12, 1, 1), jnp.float32))
    return pl.pallas_call(
        _scan_body,
        out_shape=[
            jax.ShapeDtypeStruct((_R, _C), jnp.float32),
            jax.ShapeDtypeStruct((_R, _C), jnp.float32),
        ],
        scratch_shapes=scratch,
    )(dofs_cm, pc_t)


_NW = 32
_PER_W = _N // _NW
_SLICE = _M // _NW
_CH = 32768


def _mask_last_body(kin_hbm, out_hbm, in_v, out_v):
    wid = lax.axis_index("c") * 16 + lax.axis_index("s")
    base = pl.multiple_of(wid * _PER_W, _PER_W)
    pltpu.sync_copy(kin_hbm.at[pl.ds(base, _PER_W)], in_v)

    def body(i, _):
        v = in_v[pl.ds(i * 16, 16)]
        _cnt, last = plsc.scan_count(v)
        out_v[pl.ds(i * 16, 16)] = jnp.where(last, v, _SENT)
        return 0

    lax.fori_loop(0, _PER_W // 16, body, 0)
    pltpu.sync_copy(out_v, out_hbm.at[pl.ds(base, _PER_W)])


def _scatter_score_body(mk_hbm, sn_hbm, pn_hbm, out_hbm, table, mk_v, sn_v,
                        obuf):
    wid = lax.axis_index("c") * 16 + lax.axis_index("s")
    base = pl.multiple_of(wid * _SLICE, _SLICE)
    pltpu.sync_copy(pn_hbm.at[pl.ds(base, _SLICE)], table)

    for chunk in range(_N // _CH):
        off = chunk * _CH
        pltpu.sync_copy(mk_hbm.at[pl.ds(off, _CH)], mk_v)
        pltpu.sync_copy(sn_hbm.at[pl.ds(off, _CH)], sn_v)

        def body(i, _):
            vk = mk_v[pl.ds(i * 16, 16)]
            vs = sn_v[pl.ds(i * 16, 16)]
            msk = lax.shift_right_logical(vk, 13) == wid
            loc = lax.bitwise_and(vk, _SLICE - 1)
            plsc.store_scatter(table, [loc], vs, mask=msk)
            return 0

        lax.fori_loop(0, _CH // 16, body, 0)

    def rsum(base_off):
        def rbody(i, acc):
            return acc + table[pl.ds(base_off + i * 16, 16)]
        return lax.fori_loop(0, _APP // 16, rbody,
                             jnp.zeros((16,), jnp.float32))

    s0 = jnp.sum(rsum(0))
    s1 = jnp.sum(rsum(_APP))
    io = lax.iota(jnp.int32, 16)
    obuf[...] = jnp.where(io == 0, s0, jnp.where(io == 1, s1, 0.0))
    pltpu.sync_copy(obuf, out_hbm.at[wid])


@functools.lru_cache(maxsize=1)
def _sc_kernels():
    mesh = plsc.VectorSubcoreMesh(
        core_axis_name="c", subcore_axis_name="s", num_cores=2,
        num_subcores=16)
    params = pltpu.CompilerParams(needs_layout_passes=False)
    mask_last = pl.kernel(
        _mask_last_body,
        out_type=jax.ShapeDtypeStruct((_N,), jnp.int32),
        mesh=mesh,
        compiler_params=params,
        scratch_types=[
            pltpu.VMEM((_PER_W,), jnp.int32),
            pltpu.VMEM((_PER_W,), jnp.int32),
        ],
    )
    scatter_score = pl.kernel(
        _scatter_score_body,
        out_type=jax.ShapeDtypeStruct((_NW, 16), jnp.float32),
        mesh=mesh,
        compiler_params=params,
        scratch_types=[
            pltpu.VMEM((_SLICE,), jnp.float32),
            pltpu.VMEM((_CH,), jnp.int32),
            pltpu.VMEM((_CH,), jnp.float32),
            pltpu.VMEM((16,), jnp.float32),
        ],
    )
    return mask_last, scatter_score


def kernel(full_dofs, kin_id, pose_coords):
    dofs_cm = jnp.transpose(
        full_dofs[:, :3].T.reshape(3, _C, _R), (0, 2, 1))
    pc_t = pose_coords.T.reshape(3, _R, _C)
    snorm_cm, pnorm = _scan_tc(dofs_cm, pc_t)
    snorm = snorm_cm.T.reshape(_N)
    mask_last, scatter_score = _sc_kernels()
    mkin = mask_last(kin_id)
    out = scatter_score(mkin, snorm, pnorm.reshape(_M))
    return out[:, :2].reshape(_N_POSES)
